# Initial kernel scaffold; baseline (speedup 1.0000x reference)
import jax
import jax.numpy as jnp
from jax.experimental import pallas as pl


def kernel(coords, batch, t, fourier_W, time_W, time_b, in_W, in_b, layers, out_W, out_b):
    raise NotImplementedError("write your pallas kernel here")


# speedup vs baseline: 10.7732x; 10.7732x over previous
import numpy as np
import jax
import jax.numpy as jnp
from jax.experimental import pallas as pl
from jax.experimental.pallas import tpu as pltpu

_HID = 128
_HEADS = 8
_CPH = 16
_NSLOT = 6
_R = 256
_W = 512
_BP = 128
_R2 = np.float32(0.7 * 0.7)
_F32 = jnp.float32

_G_NP = (np.arange(128)[:, None] // _CPH == np.arange(128)[None, :]).astype(np.float32)
_E_NP = _G_NP.T.copy()


def _row(v, width=128):
    v = jnp.asarray(v, _F32)
    out = jnp.zeros((8, width), _F32)
    return out.at[0, : v.shape[0]].set(v)


def _dot(a, b):
    return jnp.dot(a, b, preferred_element_type=_F32)


def _prologue_kernel(t_ref, fw_ref, twt_ref, tb_ref, mwt_ref, mb_ref,
                     lwt_ref, lb_ref, rwt_ref, rb_ref, out_ref):
    tcol = t_ref[...][:, 0:1]
    fw = fw_ref[...][0:1, :]
    xp = (2.0 * np.pi) * tcol * fw
    gfp = jnp.concatenate([jnp.sin(xp)[:, :64], jnp.cos(xp)[:, :64]], axis=1)
    tf = _dot(gfp, twt_ref[...]) + tb_ref[...][0:1, :]
    tf = tf * jax.nn.sigmoid(tf)
    for l in range(5):
        tp = _dot(tf, mwt_ref[l]) + mb_ref[l][0:1, :]
        out_ref[2 * l] = _dot(tp, lwt_ref[l]) + lb_ref[l][0:1, :]
        out_ref[2 * l + 1] = _dot(tp, rwt_ref[l]) + rb_ref[l][0:1, :]


def _knn_kernel(c0_s, nch_s, cr_ref, cc_ref, br_ref, bc_ref, inw_ref, inb_ref,
                src_ref, vm_ref, h0_ref):
    pid = pl.program_id(0)
    q = cr_ref[...]
    qx = q[:, 0:1]
    qy = q[:, 1:2]
    qz = q[:, 2:3]
    brow = br_ref[...][:, 0:1]
    rowid = pid * _R + jax.lax.broadcasted_iota(jnp.int32, (_R, 1), 0)
    c0 = c0_s[pid]
    c1 = c0 + nch_s[pid]

    chosen = []
    vals = []
    for _ in range(5):
        def body(c, carry, taken=tuple(chosen)):
            bv, bi = carry
            cc = cc_ref[c]
            cx = cc[0:1, :]
            cy = cc[1:2, :]
            cz = cc[2:3, :]
            bcol = bc_ref[c][0:1, :]
            colid = c * _W + jax.lax.broadcasted_iota(jnp.int32, (1, _W), 1)
            dx = qx - cx
            dy = qy - cy
            dz = qz - cz
            d2 = (dx * dx + dy * dy) + dz * dz
            ok = (brow == bcol) & (rowid != colid) & (d2 < _R2)
            for cj in taken:
                ok = ok & (colid != cj)
            score = jnp.where(ok, -d2, -jnp.inf)
            m = jnp.max(score, axis=1, keepdims=True)
            idx = jnp.min(
                jnp.where(score == m,
                          jnp.broadcast_to(colid, score.shape),
                          jnp.int32(2147483647)),
                axis=1, keepdims=True)
            better = (m > bv) | ((m == bv) & (idx < bi))
            return jnp.where(better, m, bv), jnp.where(better, idx, bi)

        bv0 = jnp.full((_R, 1), -jnp.inf, _F32)
        bi0 = jnp.zeros((_R, 1), jnp.int32)
        bv, bi = jax.lax.fori_loop(c0, c1, body, (bv0, bi0))
        chosen.append(bi)
        vals.append(bv)

    zi = jnp.zeros((_R, 1), jnp.int32)
    src_ref[...] = jnp.concatenate(chosen + [rowid, zi, zi], axis=1)
    ones = jnp.ones((_R, 1), _F32)
    zf = jnp.zeros((_R, 1), _F32)
    vcols = [(bv > -1.0).astype(_F32) for bv in vals]
    vm_ref[...] = jnp.concatenate(vcols + [ones, zf, zf], axis=1)
    h0_ref[...] = _dot(q, inw_ref[...]) + inb_ref[...][0:1, :]


def _proj_kernel(h_ref, br_ref, lwh_ref, rwh_ref, xlt_ref, xrt_ref,
                 xl_ref, xr_ref):
    h = h_ref[...]
    oh = (br_ref[...][:, 0:1]
          == jax.lax.broadcasted_iota(jnp.int32, (1, _BP), 1)).astype(_F32)
    xl_ref[...] = _dot(h, lwh_ref[...]) + _dot(oh, xlt_ref[...])
    xr_ref[...] = _dot(h, rwh_ref[...]) + _dot(oh, xrt_ref[...])


def _attend_kernel(c0_s, nch_s, xl_ref, xr_ref, src_ref, vm_ref, bt_ref,
                   att_ref, g_ref, e_ref, bias_ref, x_ref, seg_ref, xj_ref):
    pid = pl.program_id(0)
    src = src_ref[...]
    vm = vm_ref[...]
    xr = xr_ref[...]
    xj_ref[...] = jnp.zeros((_NSLOT, _R, 128), _F32)
    c0 = c0_s[pid]
    c1 = c0 + nch_s[pid]

    def body(c, carry):
        base = c * _W
        xlc = xl_ref[pl.ds(base, _W), :]
        colid = base + jax.lax.broadcasted_iota(jnp.int32, (1, _W), 1)
        for k in range(_NSLOT):
            oh = (src[:, k:k + 1] == colid).astype(_F32)
            xj_ref[k] += _dot(oh, xlc)
        return carry

    jax.lax.fori_loop(c0, c1, body, 0)

    att = att_ref[...][0:1, :]
    G = g_ref[...]
    E = e_ref[...]
    logits = []
    for k in range(_NSLOT):
        tv = xj_ref[k] + xr
        ev = jnp.where(tv >= 0.0, tv, 0.2 * tv)
        lg = _dot(ev * att, G)
        logits.append(jnp.where(vm[:, k:k + 1] > 0.0, lg, _F32(-1e9)))
    m = logits[0]
    for k in range(1, _NSLOT):
        m = jnp.maximum(m, logits[k])
    es = [jnp.exp(lg - m) for lg in logits]
    s = es[0]
    for k in range(1, _NSLOT):
        s = s + es[k]
    xacc = jnp.zeros((_R, 128), _F32)
    for k in range(_NSLOT):
        a = jnp.where(vm[:, k:k + 1] > 0.0, es[k] / s, 0.0)
        xacc = xacc + _dot(a, E) * xj_ref[k]
    x = xacc + bias_ref[...][0:1, :]
    x_ref[...] = x

    rs = jnp.sum(x, axis=1, keepdims=True)
    rq = jnp.sum(x * x, axis=1, keepdims=True)
    stat = jnp.concatenate(
        [rs, rq, jnp.ones((_R, 1), _F32), jnp.zeros((_R, 125), _F32)], axis=1)
    brt = bt_ref[...][0, 0:1, :]
    ohT = (jax.lax.broadcasted_iota(jnp.int32, (_BP, 1), 0) == brt).astype(_F32)

    @pl.when(pid == 0)
    def _init():
        seg_ref[...] = jnp.zeros((_BP, 128), _F32)

    seg_ref[...] += _dot(ohT, stat)


def _norm_kernel(x_ref, h_ref, br_ref, seg_ref, nw_ref, nb_ref, ho_ref):
    s = seg_ref[...]
    s1 = s[:, 0:1]
    s2 = s[:, 1:2]
    n = s[:, 2:3]
    cnt = jnp.maximum(n * _F32(_HID), 1.0)
    mean = s1 / cnt
    var = s2 / cnt - mean * mean
    inv = 1.0 / jnp.sqrt(var + 1e-5)
    par = jnp.concatenate([mean, inv, jnp.zeros((_BP, 126), _F32)], axis=1)
    oh = (br_ref[...][:, 0:1]
          == jax.lax.broadcasted_iota(jnp.int32, (1, _BP), 1)).astype(_F32)
    g = _dot(oh, par)
    x = x_ref[...]
    gn = (x - g[:, 0:1]) * g[:, 1:2] * nw_ref[...][0:1, :] + nb_ref[...][0:1, :]
    pre = gn + h_ref[...]
    ho_ref[...] = pre * jax.nn.sigmoid(pre)


def _final_kernel(h_ref, w_ref, b_ref, y_ref):
    y_ref[...] = _dot(h_ref[...], w_ref[...]) + b_ref[...][0:1, :]


def kernel(coords, batch, t, fourier_W, time_W, time_b, in_W, in_b, layers,
           out_W, out_b):
    n = coords.shape[0]
    nb = (n + _R - 1) // _R
    np_rows = nb * _R
    nc = np_rows // _W
    assert np_rows % _W == 0

    cpad = jnp.pad(coords.astype(_F32), ((0, np_rows - n), (0, 0)))
    bpad = jnp.pad(batch.astype(jnp.int32), (0, np_rows - n),
                   constant_values=_BP - 1)
    cr = jnp.pad(cpad, ((0, 0), (0, 5)))
    cc = jnp.pad(cpad.T, ((0, 5), (0, 0))).reshape(8, nc, _W).transpose(1, 0, 2)
    br = jnp.pad(bpad[:, None], ((0, 0), (0, 7)))
    bc = jnp.pad(bpad[None, :], ((0, 7), (0, 0))).reshape(8, nc, _W)
    bc = bc.transpose(1, 0, 2)
    bt = jnp.pad(bpad.reshape(nb, 1, _R), ((0, 0), (0, 7), (0, 0)))

    gids = jnp.arange(_BP, dtype=jnp.int32)
    ss = jnp.searchsorted(bpad, gids, side='left').astype(jnp.int32)
    se = jnp.searchsorted(bpad, gids, side='right').astype(jnp.int32)
    bs = jnp.arange(nb, dtype=jnp.int32) * _R
    lo = ss[bpad[bs]]
    hi = se[bpad[bs + _R - 1]]
    c0 = lo // _W
    nch = (hi + _W - 1) // _W - c0

    smem = pl.BlockSpec(memory_space=pltpu.SMEM)
    full = lambda shape: pl.BlockSpec(shape, lambda i: (0,) * len(shape))
    rowb = lambda w: pl.BlockSpec((_R, w), lambda i: (i, 0))

    t_col = jnp.pad(t.astype(_F32), (0, _BP - t.shape[0]))[:, None]
    t_col = jnp.pad(t_col, ((0, 0), (0, 7)))
    mwt = jnp.stack([l['mW'].T for l in layers])
    mb = jnp.stack([_row(l['mb']) for l in layers])
    lwt = jnp.stack([l['lW'][:, _HID:].T for l in layers])
    lb = jnp.stack([_row(l['lb']) for l in layers])
    rwt = jnp.stack([l['rW'][:, _HID:].T for l in layers])
    rb = jnp.stack([_row(l['rb']) for l in layers])
    tabs = pl.pallas_call(
        _prologue_kernel,
        out_shape=jax.ShapeDtypeStruct((10, 128, 128), _F32),
    )(t_col, _row(fourier_W), time_W.T.astype(_F32), _row(time_b),
      mwt, mb, lwt, lb, rwt, rb)

    inw = jnp.pad(in_W.T.astype(_F32), ((0, 5), (0, 0)))
    src, vm, h = pl.pallas_call(
        _knn_kernel,
        grid=(nb,),
        in_specs=[smem, smem,
                  rowb(8), full((nc, 8, _W)), rowb(8), full((nc, 8, _W)),
                  full((8, 128)), full((8, 128))],
        out_specs=[rowb(8), rowb(8), rowb(128)],
        out_shape=[jax.ShapeDtypeStruct((np_rows, 8), jnp.int32),
                   jax.ShapeDtypeStruct((np_rows, 8), _F32),
                   jax.ShapeDtypeStruct((np_rows, 128), _F32)],
    )(c0, nch, cr, cc, br, bc, inw, _row(in_b))

    Gc = jnp.asarray(_G_NP)
    Ec = jnp.asarray(_E_NP)

    for li, lyr in enumerate(layers):
        lwh = lyr['lW'][:, :_HID].T.astype(_F32)
        rwh = lyr['rW'][:, :_HID].T.astype(_F32)
        xl, xr = pl.pallas_call(
            _proj_kernel,
            grid=(nb,),
            in_specs=[rowb(128), rowb(8), full((128, 128)), full((128, 128)),
                      full((128, 128)), full((128, 128))],
            out_specs=[rowb(128), rowb(128)],
            out_shape=[jax.ShapeDtypeStruct((np_rows, 128), _F32)] * 2,
        )(h, br, lwh, rwh, tabs[2 * li], tabs[2 * li + 1])

        x, seg = pl.pallas_call(
            _attend_kernel,
            grid=(nb,),
            in_specs=[smem, smem,
                      full((np_rows, 128)), rowb(128), rowb(8), rowb(8),
                      pl.BlockSpec((1, 8, _R), lambda i: (i, 0, 0)),
                      full((8, 128)), full((128, 128)), full((128, 128)),
                      full((8, 128))],
            out_specs=[rowb(128), full((_BP, 128))],
            out_shape=[jax.ShapeDtypeStruct((np_rows, 128), _F32),
                       jax.ShapeDtypeStruct((_BP, 128), _F32)],
            scratch_shapes=[pltpu.VMEM((_NSLOT, _R, 128), _F32)],
        )(c0, nch, xl, xr, src, vm, bt,
          _row(lyr['att'].reshape(-1)), Gc, Ec, _row(lyr['bias']))

        h = pl.pallas_call(
            _norm_kernel,
            grid=(nb,),
            in_specs=[rowb(128), rowb(128), rowb(8), full((_BP, 128)),
                      full((8, 128)), full((8, 128))],
            out_specs=rowb(128),
            out_shape=jax.ShapeDtypeStruct((np_rows, 128), _F32),
        )(x, h, br, seg, _row(lyr['nw']), _row(lyr['nb']))

    outw = jnp.pad(out_W.T.astype(_F32), ((0, 0), (0, 128 - out_W.shape[0])))
    y = pl.pallas_call(
        _final_kernel,
        grid=(nb,),
        in_specs=[rowb(128), full((128, 128)), full((8, 128))],
        out_specs=rowb(128),
        out_shape=jax.ShapeDtypeStruct((np_rows, 128), _F32),
    )(h, outw, _row(out_b))
    return y[:n, :out_W.shape[0]]


# speedup vs baseline: 14.4821x; 1.3443x over previous
import functools

import numpy as np
import jax
import jax.numpy as jnp
from jax.experimental import pallas as pl
from jax.experimental.pallas import tpu as pltpu

_HID = 128
_CPH = 16
_NSLOT = 6
_R = 256
_W = 512
_BP = 128
_R2 = np.float32(0.7 * 0.7)
_F32 = jnp.float32
_IMAX = np.int32(2147483647)

_G_NP = (np.arange(128)[:, None] // _CPH == np.arange(128)[None, :]).astype(np.float32)
_E_NP = _G_NP.T.copy()


def _row(v, width=128):
    v = jnp.asarray(v, _F32)
    out = jnp.zeros((8, width), _F32)
    return out.at[0, : v.shape[0]].set(v)


def _dot(a, b):
    return jnp.dot(a, b, preferred_element_type=_F32)


def _onehot(brow):
    return (brow == jax.lax.broadcasted_iota(jnp.int32, (1, _BP), 1)).astype(_F32)


def _prologue_kernel(t_ref, fw_ref, twt_ref, tb_ref, mwt_ref, mb_ref,
                     lwt_ref, lb_ref, rwt_ref, rb_ref, out_ref):
    tcol = t_ref[...][:, 0:1]
    fw = fw_ref[...][0:1, :]
    xp = (2.0 * np.pi) * tcol * fw
    gfp = jnp.concatenate([jnp.sin(xp)[:, :64], jnp.cos(xp)[:, :64]], axis=1)
    tf = _dot(gfp, twt_ref[...]) + tb_ref[...][0:1, :]
    tf = tf * jax.nn.sigmoid(tf)
    for l in range(5):
        tp = _dot(tf, mwt_ref[l]) + mb_ref[l][0:1, :]
        out_ref[2 * l] = _dot(tp, lwt_ref[l]) + lb_ref[l][0:1, :]
        out_ref[2 * l + 1] = _dot(tp, rwt_ref[l]) + rb_ref[l][0:1, :]


def _knn_kernel(c0_s, nch_s, cr_ref, cc_ref, br_ref, bc_ref, inw_ref, inb_ref,
                lwh_ref, rwh_ref, xlt_ref, xrt_ref,
                src_ref, vm_ref, h0_ref, xl_ref, xr_ref):
    pid = pl.program_id(0)
    q = cr_ref[...]
    qx = q[:, 0:1]
    qy = q[:, 1:2]
    qz = q[:, 2:3]
    brow = br_ref[...][:, 0:1]
    rowid = pid * _R + jax.lax.broadcasted_iota(jnp.int32, (_R, 1), 0)
    c0 = c0_s[pid]
    c1 = c0 + nch_s[pid]

    def body(c, carry):
        bvs = list(carry[:5])
        bis = list(carry[5:])
        cc = cc_ref[c]
        cx = cc[0:1, :]
        cy = cc[1:2, :]
        cz = cc[2:3, :]
        bcol = bc_ref[c][0:1, :]
        colid = c * _W + jax.lax.broadcasted_iota(jnp.int32, (1, _W), 1)
        dx = qx - cx
        dy = qy - cy
        dz = qz - cz
        d2 = (dx * dx + dy * dy) + dz * dz
        ok = (brow == bcol) & (rowid != colid) & (d2 < _R2)
        score = jnp.where(ok, -d2, -jnp.inf)
        for _ in range(5):
            m = jnp.max(score, axis=1, keepdims=True)
            idx = jnp.min(jnp.where(score == m, colid, _IMAX),
                          axis=1, keepdims=True)
            score = jnp.where(colid == idx, -jnp.inf, score)
            bvs.append(m)
            bis.append(idx)
        padv = jnp.full((_R, 6), -jnp.inf, _F32)
        padi = jnp.full((_R, 6), _IMAX, jnp.int32)
        cv = jnp.concatenate(bvs + [padv], axis=1)
        ci = jnp.concatenate(bis + [padi], axis=1)
        nbv = []
        nbi = []
        for _ in range(5):
            m = jnp.max(cv, axis=1, keepdims=True)
            idx = jnp.min(jnp.where(cv == m, ci, _IMAX), axis=1, keepdims=True)
            cv = jnp.where((cv == m) & (ci == idx), -jnp.inf, cv)
            nbv.append(m)
            nbi.append(idx)
        return tuple(nbv) + tuple(nbi)

    init = tuple(jnp.full((_R, 1), -jnp.inf, _F32) for _ in range(5)) + \
           tuple(jnp.full((_R, 1), _IMAX, jnp.int32) for _ in range(5))
    carry = jax.lax.fori_loop(c0, c1, body, init)
    vals = carry[:5]
    chosen = carry[5:]

    zi = jnp.zeros((_R, 1), jnp.int32)
    src_ref[...] = jnp.concatenate(list(chosen) + [rowid, zi, zi], axis=1)
    ones = jnp.ones((_R, 1), _F32)
    zf = jnp.zeros((_R, 1), _F32)
    vcols = [(bv > -1.0).astype(_F32) for bv in vals]
    vm_ref[...] = jnp.concatenate(vcols + [ones, zf, zf], axis=1)

    h0 = _dot(q, inw_ref[...]) + inb_ref[...][0:1, :]
    h0_ref[...] = h0
    oh = _onehot(brow)
    xl_ref[...] = _dot(h0, lwh_ref[...]) + _dot(oh, xlt_ref[...])
    xr_ref[...] = _dot(h0, rwh_ref[...]) + _dot(oh, xrt_ref[...])


def _attend_kernel(np_rows, lo8_s, nch_s, xl_ref, xr_ref, src_ref, vm_ref,
                   bt_ref, att_ref, g_ref, e_ref, bias_ref,
                   x_ref, seg_ref, xj_ref):
    pid = pl.program_id(0)
    src = src_ref[...]
    vm = vm_ref[...]
    xr = xr_ref[...]
    xj_ref[...] = jnp.zeros((_NSLOT, _R, 128), _F32)
    lo8 = lo8_s[pid]
    c1 = nch_s[pid]

    def body(c, carry):
        nom_s = lo8 + c * _W
        start = jnp.minimum(nom_s, np_rows - _W)
        xlc = xl_ref[pl.ds(start, _W), :]
        colid = start + jax.lax.broadcasted_iota(jnp.int32, (1, _W), 1)
        inr = (colid >= nom_s) & (colid < nom_s + _W)
        mcol = jnp.where(inr, colid, -1)
        for k in range(_NSLOT):
            oh = (src[:, k:k + 1] == mcol).astype(_F32)
            xj_ref[k] += _dot(oh, xlc)
        return carry

    jax.lax.fori_loop(0, c1, body, 0)

    att = att_ref[...][0:1, :]
    G = g_ref[...]
    E = e_ref[...]
    logits = []
    for k in range(_NSLOT):
        tv = xj_ref[k] + xr
        ev = jnp.where(tv >= 0.0, tv, 0.2 * tv)
        lg = _dot(ev * att, G)
        logits.append(jnp.where(vm[:, k:k + 1] > 0.0, lg, _F32(-1e9)))
    m = logits[0]
    for k in range(1, _NSLOT):
        m = jnp.maximum(m, logits[k])
    es = [jnp.exp(lg - m) for lg in logits]
    s = es[0]
    for k in range(1, _NSLOT):
        s = s + es[k]
    xacc = jnp.zeros((_R, 128), _F32)
    for k in range(_NSLOT):
        a = jnp.where(vm[:, k:k + 1] > 0.0, es[k] / s, 0.0)
        xacc = xacc + _dot(a, E) * xj_ref[k]
    x = xacc + bias_ref[...][0:1, :]
    x_ref[...] = x

    rs = jnp.sum(x, axis=1, keepdims=True)
    rq = jnp.sum(x * x, axis=1, keepdims=True)
    stat = jnp.concatenate(
        [rs, rq, jnp.ones((_R, 1), _F32), jnp.zeros((_R, 125), _F32)], axis=1)
    brt = bt_ref[...][0, 0:1, :]
    ohT = (jax.lax.broadcasted_iota(jnp.int32, (_BP, 1), 0) == brt).astype(_F32)

    @pl.when(pid == 0)
    def _init():
        seg_ref[...] = jnp.zeros((_BP, 128), _F32)

    seg_ref[...] += _dot(ohT, stat)


def _norm_common(x_ref, h_ref, br_ref, seg_ref, nw_ref, nb_ref):
    s = seg_ref[...]
    s1 = s[:, 0:1]
    s2 = s[:, 1:2]
    n = s[:, 2:3]
    cnt = jnp.maximum(n * _F32(_HID), 1.0)
    mean = s1 / cnt
    var = s2 / cnt - mean * mean
    inv = 1.0 / jnp.sqrt(var + 1e-5)
    par = jnp.concatenate([mean, inv, jnp.zeros((_BP, 126), _F32)], axis=1)
    oh = _onehot(br_ref[...][:, 0:1])
    g = _dot(oh, par)
    x = x_ref[...]
    gn = (x - g[:, 0:1]) * g[:, 1:2] * nw_ref[...][0:1, :] + nb_ref[...][0:1, :]
    pre = gn + h_ref[...]
    return pre * jax.nn.sigmoid(pre), oh


def _normproj_kernel(x_ref, h_ref, br_ref, seg_ref, nw_ref, nb_ref,
                     lwh_ref, rwh_ref, xlt_ref, xrt_ref,
                     ho_ref, xl_ref, xr_ref):
    hn, oh = _norm_common(x_ref, h_ref, br_ref, seg_ref, nw_ref, nb_ref)
    ho_ref[...] = hn
    xl_ref[...] = _dot(hn, lwh_ref[...]) + _dot(oh, xlt_ref[...])
    xr_ref[...] = _dot(hn, rwh_ref[...]) + _dot(oh, xrt_ref[...])


def _normfinal_kernel(x_ref, h_ref, br_ref, seg_ref, nw_ref, nb_ref,
                      ow_ref, ob_ref, y_ref):
    hn, _ = _norm_common(x_ref, h_ref, br_ref, seg_ref, nw_ref, nb_ref)
    y_ref[...] = _dot(hn, ow_ref[...]) + ob_ref[...][0:1, :]


def kernel(coords, batch, t, fourier_W, time_W, time_b, in_W, in_b, layers,
           out_W, out_b):
    n = coords.shape[0]
    nb = (n + _R - 1) // _R
    np_rows = nb * _R
    nc = np_rows // _W
    assert np_rows % _W == 0

    cpad = jnp.pad(coords.astype(_F32), ((0, np_rows - n), (0, 0)))
    bpad = jnp.pad(batch.astype(jnp.int32), (0, np_rows - n),
                   constant_values=_BP - 1)
    cr = jnp.pad(cpad, ((0, 0), (0, 5)))
    cc = jnp.pad(cpad.T, ((0, 5), (0, 0))).reshape(8, nc, _W).transpose(1, 0, 2)
    br = jnp.pad(bpad[:, None], ((0, 0), (0, 7)))
    bc = jnp.pad(bpad[None, :], ((0, 7), (0, 0))).reshape(8, nc, _W)
    bc = bc.transpose(1, 0, 2)
    bt = jnp.pad(bpad.reshape(nb, 1, _R), ((0, 0), (0, 7), (0, 0)))

    gids = jnp.arange(_BP, dtype=jnp.int32)
    ss = jnp.searchsorted(bpad, gids, side='left').astype(jnp.int32)
    se = jnp.searchsorted(bpad, gids, side='right').astype(jnp.int32)
    bs = jnp.arange(nb, dtype=jnp.int32) * _R
    lo = ss[bpad[bs]]
    hi = se[bpad[bs + _R - 1]]
    c0 = lo // _W
    nch = (hi + _W - 1) // _W - c0
    lo8 = (lo // 8) * 8
    ncha = (hi - lo8 + _W - 1) // _W

    smem = pl.BlockSpec(memory_space=pltpu.SMEM)
    full = lambda shape: pl.BlockSpec(shape, lambda i: (0,) * len(shape))
    rowb = lambda w: pl.BlockSpec((_R, w), lambda i: (i, 0))

    t_col = jnp.pad(t.astype(_F32), (0, _BP - t.shape[0]))[:, None]
    t_col = jnp.pad(t_col, ((0, 0), (0, 7)))
    mwt = jnp.stack([l['mW'].T for l in layers])
    mb = jnp.stack([_row(l['mb']) for l in layers])
    lwt = jnp.stack([l['lW'][:, _HID:].T for l in layers])
    lb = jnp.stack([_row(l['lb']) for l in layers])
    rwt = jnp.stack([l['rW'][:, _HID:].T for l in layers])
    rb = jnp.stack([_row(l['rb']) for l in layers])
    tabs = pl.pallas_call(
        _prologue_kernel,
        out_shape=jax.ShapeDtypeStruct((10, 128, 128), _F32),
    )(t_col, _row(fourier_W), time_W.T.astype(_F32), _row(time_b),
      mwt, mb, lwt, lb, rwt, rb)

    lwh = [l['lW'][:, :_HID].T.astype(_F32) for l in layers]
    rwh = [l['rW'][:, :_HID].T.astype(_F32) for l in layers]

    inw = jnp.pad(in_W.T.astype(_F32), ((0, 5), (0, 0)))
    src, vm, h, xl, xr = pl.pallas_call(
        _knn_kernel,
        grid=(nb,),
        in_specs=[smem, smem,
                  rowb(8), full((nc, 8, _W)), rowb(8), full((nc, 8, _W)),
                  full((8, 128)), full((8, 128)), full((128, 128)),
                  full((128, 128)), full((128, 128)), full((128, 128))],
        out_specs=[rowb(8), rowb(8), rowb(128), rowb(128), rowb(128)],
        out_shape=[jax.ShapeDtypeStruct((np_rows, 8), jnp.int32),
                   jax.ShapeDtypeStruct((np_rows, 8), _F32)]
                  + [jax.ShapeDtypeStruct((np_rows, 128), _F32)] * 3,
    )(c0, nch, cr, cc, br, bc, inw, _row(in_b), lwh[0], rwh[0],
      tabs[0], tabs[1])

    Gc = jnp.asarray(_G_NP)
    Ec = jnp.asarray(_E_NP)

    for li, lyr in enumerate(layers):
        x, seg = pl.pallas_call(
            functools.partial(_attend_kernel, np_rows),
            grid=(nb,),
            in_specs=[smem, smem,
                      full((np_rows, 128)), rowb(128), rowb(8), rowb(8),
                      pl.BlockSpec((1, 8, _R), lambda i: (i, 0, 0)),
                      full((8, 128)), full((128, 128)), full((128, 128)),
                      full((8, 128))],
            out_specs=[rowb(128), full((_BP, 128))],
            out_shape=[jax.ShapeDtypeStruct((np_rows, 128), _F32),
                       jax.ShapeDtypeStruct((_BP, 128), _F32)],
            scratch_shapes=[pltpu.VMEM((_NSLOT, _R, 128), _F32)],
        )(lo8, ncha, xl, xr, src, vm, bt,
          _row(lyr['att'].reshape(-1)), Gc, Ec, _row(lyr['bias']))

        if li < 4:
            nxt = layers[li + 1]
            h, xl, xr = pl.pallas_call(
                _normproj_kernel,
                grid=(nb,),
                in_specs=[rowb(128), rowb(128), rowb(8), full((_BP, 128)),
                          full((8, 128)), full((8, 128)), full((128, 128)),
                          full((128, 128)), full((128, 128)), full((128, 128))],
                out_specs=[rowb(128)] * 3,
                out_shape=[jax.ShapeDtypeStruct((np_rows, 128), _F32)] * 3,
            )(x, h, br, seg, _row(lyr['nw']), _row(lyr['nb']),
              lwh[li + 1], rwh[li + 1], tabs[2 * li + 2], tabs[2 * li + 3])
        else:
            outw = jnp.pad(out_W.T.astype(_F32),
                           ((0, 0), (0, 128 - out_W.shape[0])))
            y = pl.pallas_call(
                _normfinal_kernel,
                grid=(nb,),
                in_specs=[rowb(128), rowb(128), rowb(8), full((_BP, 128)),
                          full((8, 128)), full((8, 128)), full((128, 128)),
                          full((8, 128))],
                out_specs=rowb(128),
                out_shape=jax.ShapeDtypeStruct((np_rows, 128), _F32),
            )(x, h, br, seg, _row(lyr['nw']), _row(lyr['nb']),
              outw, _row(out_b))
    return y[:n, :out_W.shape[0]]
